# baseline (device time: 85562 ns/iter reference)
import jax
import jax.numpy as jnp
from jax import lax
from jax.experimental import pallas as pl
from jax.experimental.pallas import tpu as pltpu

B, S, H, Dh, Dr = 2, 512, 16, 128, 32
D = 2048
DC = 128
BS = B * S
SCALE = (Dh + Dr) ** -0.5
BF16 = jnp.bfloat16
NT = 256
NSTRIP = D // NT
KVT = 512


def _mm(a, b):
    return lax.dot_general(a, b, (((1,), (0,)), ((), ())),
                           preferred_element_type=jnp.float32)


def _mmT(a, b):
    return lax.dot_general(a, b, (((1,), (1,)), ((), ())),
                           preferred_element_type=jnp.float32)


def kernel(x, Wdkv, Wuk, Wuv, Wq, Wqr, Wkr, Wo):
    def body(x_ref, wdkv_ref, wuk_ref, wuv_ref, wq_hbm, wqr_ref, wkr_ref,
             wo_hbm, out_ref,
             xb_ref, c_self, c_other, w_send, w_other,
             q_ref, qr_ref, kr_ref, k_ref, v_ref, strip_buf, wo_buf,
             send_sems, recv_sems, strip_sems, wo_sems):
        my_x = lax.axis_index("x")
        my_y = lax.axis_index("y")
        my_z = lax.axis_index("z")
        partner = (1 - my_x, my_y, my_z)

        def strip_cp(hbm_ref, n):
            return pltpu.make_async_copy(
                hbm_ref.at[:, pl.ds(n * NT, NT)],
                strip_buf.at[n % 2], strip_sems.at[n % 2])

        strip_cp(wq_hbm, 0).start()

        barrier = pltpu.get_barrier_semaphore()
        pl.semaphore_signal(barrier, inc=1, device_id=partner,
                            device_id_type=pl.DeviceIdType.MESH)
        pl.semaphore_wait(barrier, 1)

        w_send[0] = wuk_ref[:].astype(BF16)
        w_send[1] = wuv_ref[:].astype(BF16)
        w_rdma = pltpu.make_async_remote_copy(
            src_ref=w_send, dst_ref=w_other,
            send_sem=send_sems.at[0], recv_sem=recv_sems.at[0],
            device_id=partner, device_id_type=pl.DeviceIdType.MESH)
        w_rdma.start()

        wdkv = wdkv_ref[:].astype(BF16)
        for b in range(B):
            xb_ref[pl.ds(b * S, S), :] = x_ref[b].astype(BF16)
            c_self[pl.ds(b * S, S), :] = _mm(
                xb_ref[pl.ds(b * S, S), :], wdkv).astype(BF16)
        c_rdma = pltpu.make_async_remote_copy(
            src_ref=c_self, dst_ref=c_other,
            send_sem=send_sems.at[1], recv_sem=recv_sems.at[1],
            device_id=partner, device_id_type=pl.DeviceIdType.MESH)
        c_rdma.start()

        for n in range(NSTRIP):
            if n + 1 < NSTRIP:
                strip_cp(wq_hbm, n + 1).start()
            strip_cp(wq_hbm, n).wait()
            wqs = strip_buf[n % 2].astype(BF16)
            for b in range(B):
                q_ref[pl.ds(b * S, S), pl.ds(n * NT, NT)] = (
                    _mm(xb_ref[pl.ds(b * S, S), :], wqs) * SCALE
                ).astype(BF16)

        wqr = wqr_ref[:].astype(BF16)
        wkr = wkr_ref[:].astype(BF16)
        for b in range(B):
            xb = xb_ref[pl.ds(b * S, S), :]
            qr_ref[pl.ds(b * S, S), :] = (
                _mm(xb, wqr) * SCALE).astype(BF16)
            kr_ref[pl.ds(b * S, S), :] = _mm(xb, wkr).astype(BF16)

        w_rdma.wait()
        c_rdma.wait()

        for b in range(B):
            cs = c_self[pl.ds(b * S, S), :]
            co = c_other[pl.ds(b * S, S), :]
            for n0 in range(0, D, KVT):
                cols = pl.ds(n0, KVT)
                k_ref[pl.ds(b * S, S), cols] = (
                    _mm(cs, w_send[0, :, cols])
                    + _mm(co, w_other[0, :, cols])).astype(BF16)
                v_ref[pl.ds(b * S, S), cols] = (
                    _mm(cs, w_send[1, :, cols])
                    + _mm(co, w_other[1, :, cols])).astype(BF16)

        def wo_cp(h):
            return pltpu.make_async_copy(
                wo_hbm.at[pl.ds(h * Dh, Dh), :],
                wo_buf.at[h % 2], wo_sems.at[h % 2])

        wo_cp(0).start()
        for h in range(H):
            if h + 1 < H:
                wo_cp(h + 1).start()
            wo_cp(h).wait()
            wo_h = wo_buf[h % 2].astype(BF16)
            co = h * Dh
            for b in range(B):
                ro = b * S
                q = q_ref[pl.ds(ro, S), pl.ds(co, Dh)]
                k = k_ref[pl.ds(ro, S), pl.ds(co, Dh)]
                qr = qr_ref[pl.ds(ro, S), h * Dr:(h + 1) * Dr]
                kr = kr_ref[pl.ds(ro, S), :]
                p = jnp.exp(_mmT(q, k) + _mmT(qr, kr))
                r = 1.0 / jnp.sum(p, axis=1, keepdims=True)
                o = _mm(p.astype(BF16), v_ref[pl.ds(ro, S), pl.ds(co, Dh)])
                ob = (o * r).astype(BF16)
                for n0 in range(0, D, D // 2):
                    cols = pl.ds(n0, D // 2)
                    contrib = _mm(ob, wo_h[:, n0:n0 + D // 2])
                    if h == 0:
                        out_ref[b, :, cols] = contrib
                    else:
                        out_ref[b, :, cols] = out_ref[b, :, cols] + contrib

    vmem = pl.BlockSpec(memory_space=pltpu.VMEM)
    hbm = pl.BlockSpec(memory_space=pl.ANY)
    return pl.pallas_call(
        body,
        out_shape=jax.ShapeDtypeStruct((B, S, D), jnp.float32),
        in_specs=[vmem, vmem, vmem, vmem, hbm, vmem, vmem, hbm],
        out_specs=vmem,
        scratch_shapes=[
            pltpu.VMEM((BS, D), BF16),
            pltpu.VMEM((BS, DC), BF16),
            pltpu.VMEM((BS, DC), BF16),
            pltpu.VMEM((2, DC, D), BF16),
            pltpu.VMEM((2, DC, D), BF16),
            pltpu.VMEM((BS, D), BF16),
            pltpu.VMEM((BS, H * Dr), BF16),
            pltpu.VMEM((BS, Dr), BF16),
            pltpu.VMEM((BS, D), BF16),
            pltpu.VMEM((BS, D), BF16),
            pltpu.VMEM((2, D, NT), jnp.float32),
            pltpu.VMEM((2, Dh, D), jnp.float32),
            pltpu.SemaphoreType.DMA((2,)),
            pltpu.SemaphoreType.DMA((2,)),
            pltpu.SemaphoreType.DMA((2,)),
            pltpu.SemaphoreType.DMA((2,)),
        ],
        compiler_params=pltpu.CompilerParams(
            collective_id=0,
            vmem_limit_bytes=61 * 1024 * 1024,
        ),
    )(x, Wdkv, Wuk, Wuv, Wq, Wqr, Wkr, Wo)


# device time: 69299 ns/iter; 1.2347x vs baseline; 1.2347x over previous
import os

import jax
import jax.numpy as jnp
from jax import lax
from jax.experimental import pallas as pl
from jax.experimental.pallas import tpu as pltpu

B, S, H, Dh, Dr = 2, 512, 16, 128, 32
D = 2048
DC = 128
BS = B * S
SCALE = (Dh + Dr) ** -0.5
BF16 = jnp.bfloat16
_ABL = set(os.environ.get("KABL", "").split(","))
NT = 256
NSTRIP = D // NT
KVT = 512


def _mm(a, b):
    return lax.dot_general(a, b, (((1,), (0,)), ((), ())),
                           preferred_element_type=jnp.float32)


def _mmT(a, b):
    return lax.dot_general(a, b, (((1,), (1,)), ((), ())),
                           preferred_element_type=jnp.float32)


def kernel(x, Wdkv, Wuk, Wuv, Wq, Wqr, Wkr, Wo):
    def body(x_ref, wdkv_ref, wuk_ref, wuv_ref, wq_hbm, wqr_ref, wkr_ref,
             wo_hbm, out_ref,
             xb_ref, c_self, c_other, w_send, w_other,
             q_ref, qr_ref, kr_ref, k_ref, v_ref, strip_buf, wo_buf,
             ob_buf, send_sems, recv_sems, strip_sems, wo_sems):
        my_x = lax.axis_index("x")
        my_y = lax.axis_index("y")
        my_z = lax.axis_index("z")
        partner = (1 - my_x, my_y, my_z)

        def strip_cp(hbm_ref, n):
            return pltpu.make_async_copy(
                hbm_ref.at[:, pl.ds(n * NT, NT)],
                strip_buf.at[n % 2], strip_sems.at[n % 2])

        strip_cp(wq_hbm, 0).start()

        barrier = pltpu.get_barrier_semaphore()
        pl.semaphore_signal(barrier, inc=1, device_id=partner,
                            device_id_type=pl.DeviceIdType.MESH)
        pl.semaphore_wait(barrier, 1)

        with jax.named_scope("phase#send"):
            w_send[0] = wuk_ref[:].astype(BF16)
            w_send[1] = wuv_ref[:].astype(BF16)
            w_rdma = pltpu.make_async_remote_copy(
                src_ref=w_send, dst_ref=w_other,
                send_sem=send_sems.at[0], recv_sem=recv_sems.at[0],
                device_id=partner, device_id_type=pl.DeviceIdType.MESH)
            w_rdma.start()

            wdkv = wdkv_ref[:].astype(BF16)
            for b in range(B):
                xb_ref[pl.ds(b * S, S), :] = x_ref[b].astype(BF16)
                c_self[pl.ds(b * S, S), :] = _mm(
                    xb_ref[pl.ds(b * S, S), :], wdkv).astype(BF16)
            c_rdma = pltpu.make_async_remote_copy(
                src_ref=c_self, dst_ref=c_other,
                send_sem=send_sems.at[1], recv_sem=recv_sems.at[1],
                device_id=partner, device_id_type=pl.DeviceIdType.MESH)
            c_rdma.start()

        with jax.named_scope("phase#qstream"):
            for n in range(NSTRIP if "qstream" not in _ABL else 0):
                if n + 1 < NSTRIP:
                    strip_cp(wq_hbm, n + 1).start()
                strip_cp(wq_hbm, n).wait()
                wqs = strip_buf[n % 2].astype(BF16)
                for b in range(B):
                    q_ref[pl.ds(b * S, S), pl.ds(n * NT, NT)] = (
                        _mm(xb_ref[pl.ds(b * S, S), :], wqs) * SCALE
                    ).astype(BF16)

        with jax.named_scope("phase#qr_kr"):
            wqr = wqr_ref[:].astype(BF16)
            wkr = wkr_ref[:].astype(BF16)
            for b in range(B):
                xb = xb_ref[pl.ds(b * S, S), :]
                qr_ref[pl.ds(b * S, S), :] = (
                    _mm(xb, wqr) * SCALE).astype(BF16)
                kr_ref[pl.ds(b * S, S), :] = _mm(xb, wkr).astype(BF16)

        with jax.named_scope("phase#rdmawait"):
            w_rdma.wait()
            c_rdma.wait()

        with jax.named_scope("phase#kv"):
            for b in range(B if "kv" not in _ABL else 0):
                cs = c_self[pl.ds(b * S, S), :]
                co = c_other[pl.ds(b * S, S), :]
                for n0 in range(0, D, KVT):
                    cols = pl.ds(n0, KVT)
                    k_ref[pl.ds(b * S, S), cols] = (
                        _mm(cs, w_send[0, :, cols])
                        + _mm(co, w_other[0, :, cols])).astype(BF16)
                    v_ref[pl.ds(b * S, S), cols] = (
                        _mm(cs, w_send[1, :, cols])
                        + _mm(co, w_other[1, :, cols])).astype(BF16)

        GH = 4
        NG = H // GH
        GD = GH * Dh

        def wo_cp(g):
            return pltpu.make_async_copy(
                wo_hbm.at[pl.ds(g * GD, GD), :],
                wo_buf.at[g % 2], wo_sems.at[g % 2])

        if "attnout" in _ABL:
            for b in range(B):
                out_ref[b] = jnp.zeros((S, D), jnp.float32)
        else:
            wo_cp(0).start()
        with jax.named_scope("phase#attnout"):
            for g in range(NG if "attnout" not in _ABL else 0):
                if g + 1 < NG:
                    wo_cp(g + 1).start()
                for hl in range(GH):
                    h = g * GH + hl
                    co = h * Dh
                    for b in range(B):
                        ro = b * S
                        q = q_ref[pl.ds(ro, S), pl.ds(co, Dh)]
                        k = k_ref[pl.ds(ro, S), pl.ds(co, Dh)]
                        qr = qr_ref[pl.ds(ro, S), h * Dr:(h + 1) * Dr]
                        kr = kr_ref[pl.ds(ro, S), :]
                        p = jnp.exp(_mmT(q, k) + _mmT(qr, kr))
                        r = 1.0 / jnp.sum(p, axis=1, keepdims=True)
                        o = _mm(p.astype(BF16),
                                v_ref[pl.ds(ro, S), pl.ds(co, Dh)])
                        ob_buf[pl.ds(ro, S), pl.ds(hl * Dh, Dh)] = (
                            o * r).astype(BF16)
                wo_cp(g).wait()
                for n0 in range(0, D, D // 2):
                    woh = wo_buf[g % 2][:, n0:n0 + D // 2].astype(BF16)
                    cols = pl.ds(n0, D // 2)
                    for b in range(B):
                        contrib = _mm(ob_buf[pl.ds(b * S, S), :], woh)
                        if g == 0:
                            out_ref[b, :, cols] = contrib
                        else:
                            out_ref[b, :, cols] = (
                                out_ref[b, :, cols] + contrib)

    vmem = pl.BlockSpec(memory_space=pltpu.VMEM)
    hbm = pl.BlockSpec(memory_space=pl.ANY)
    return pl.pallas_call(
        body,
        out_shape=jax.ShapeDtypeStruct((B, S, D), jnp.float32),
        in_specs=[vmem, vmem, vmem, vmem, hbm, vmem, vmem, hbm],
        out_specs=vmem,
        scratch_shapes=[
            pltpu.VMEM((BS, D), BF16),
            pltpu.VMEM((BS, DC), BF16),
            pltpu.VMEM((BS, DC), BF16),
            pltpu.VMEM((2, DC, D), BF16),
            pltpu.VMEM((2, DC, D), BF16),
            pltpu.VMEM((BS, D), BF16),
            pltpu.VMEM((BS, H * Dr), BF16),
            pltpu.VMEM((BS, Dr), BF16),
            pltpu.VMEM((BS, D), BF16),
            pltpu.VMEM((BS, D), BF16),
            pltpu.VMEM((2, D, NT), jnp.float32),
            pltpu.VMEM((2, 4 * Dh, D), jnp.float32),
            pltpu.VMEM((BS, 4 * Dh), BF16),
            pltpu.SemaphoreType.DMA((2,)),
            pltpu.SemaphoreType.DMA((2,)),
            pltpu.SemaphoreType.DMA((2,)),
            pltpu.SemaphoreType.DMA((2,)),
        ],
        compiler_params=pltpu.CompilerParams(
            collective_id=0,
            vmem_limit_bytes=61 * 1024 * 1024,
        ),
    )(x, Wdkv, Wuk, Wuv, Wq, Wqr, Wkr, Wo)


# device time: 67110 ns/iter; 1.2750x vs baseline; 1.0326x over previous
import os

import jax
import jax.numpy as jnp
from jax import lax
from jax.experimental import pallas as pl
from jax.experimental.pallas import tpu as pltpu

B, S, H, Dh, Dr = 2, 512, 16, 128, 32
D = 2048
DC = 128
BS = B * S
SCALE = (Dh + Dr) ** -0.5
BF16 = jnp.bfloat16
_ABL = set(os.environ.get("KABL", "").split(","))
NT = 256
NSTRIP = D // NT
KVT = 512


def _mm(a, b):
    return lax.dot_general(a, b, (((1,), (0,)), ((), ())),
                           preferred_element_type=jnp.float32)


def _mmT(a, b):
    return lax.dot_general(a, b, (((1,), (1,)), ((), ())),
                           preferred_element_type=jnp.float32)


def kernel(x, Wdkv, Wuk, Wuv, Wq, Wqr, Wkr, Wo):
    def body(x_ref, wdkv_ref, wuk_ref, wuv_ref, wq_hbm, wqr_ref, wkr_ref,
             wo_hbm, out_ref,
             xb_ref, c_full, w_kv,
             q_ref, qr_ref, kr_ref, k_ref, v_ref, strip_buf, wo_buf,
             ob_buf, send_sems, recv_sems, strip_sems, wo_sems):
        my_x = lax.axis_index("x")
        my_y = lax.axis_index("y")
        my_z = lax.axis_index("z")
        partner = (1 - my_x, my_y, my_z)

        def strip_cp(hbm_ref, n):
            return pltpu.make_async_copy(
                hbm_ref.at[:, pl.ds(n * NT, NT)],
                strip_buf.at[n % 2], strip_sems.at[n % 2])

        if "qstream" not in _ABL:
            strip_cp(wq_hbm, 0).start()

        barrier = pltpu.get_barrier_semaphore()
        pl.semaphore_signal(barrier, inc=1, device_id=partner,
                            device_id_type=pl.DeviceIdType.MESH)
        pl.semaphore_wait(barrier, 1)

        with jax.named_scope("phase#send"):
            w_kv[0, :DC, :] = wuk_ref[:].astype(BF16)
            w_kv[1, :DC, :] = wuv_ref[:].astype(BF16)
            w_rdma = pltpu.make_async_remote_copy(
                src_ref=w_kv.at[:, pl.ds(0, DC), :],
                dst_ref=w_kv.at[:, pl.ds(DC, DC), :],
                send_sem=send_sems.at[0], recv_sem=recv_sems.at[0],
                device_id=partner, device_id_type=pl.DeviceIdType.MESH)
            w_rdma.start()

            wdkv = wdkv_ref[:].astype(BF16)
            for b in range(B):
                xb_ref[pl.ds(b * S, S), :] = x_ref[b].astype(BF16)
                c_full[pl.ds(b * S, S), :DC] = _mm(
                    xb_ref[pl.ds(b * S, S), :], wdkv).astype(BF16)
            c_rdma = pltpu.make_async_remote_copy(
                src_ref=c_full.at[:, pl.ds(0, DC)],
                dst_ref=c_full.at[:, pl.ds(DC, DC)],
                send_sem=send_sems.at[1], recv_sem=recv_sems.at[1],
                device_id=partner, device_id_type=pl.DeviceIdType.MESH)
            c_rdma.start()

        with jax.named_scope("phase#qstream"):
            for n in range(NSTRIP if "qstream" not in _ABL else 0):
                if n + 1 < NSTRIP:
                    strip_cp(wq_hbm, n + 1).start()
                strip_cp(wq_hbm, n).wait()
                wqs = strip_buf[n % 2].astype(BF16)
                for b in range(B):
                    q_ref[pl.ds(b * S, S), pl.ds(n * NT, NT)] = (
                        _mm(xb_ref[pl.ds(b * S, S), :], wqs) * SCALE
                    ).astype(BF16)

        with jax.named_scope("phase#qr_kr"):
            wqr = wqr_ref[:].astype(BF16)
            wkr = wkr_ref[:].astype(BF16)
            for b in range(B):
                xb = xb_ref[pl.ds(b * S, S), :]
                qr_ref[pl.ds(b * S, S), :] = (
                    _mm(xb, wqr) * SCALE).astype(BF16)
                kr_ref[pl.ds(b * S, S), :] = _mm(xb, wkr).astype(BF16)

        with jax.named_scope("phase#rdmawait"):
            w_rdma.wait()
            c_rdma.wait()

        with jax.named_scope("phase#kv"):
            for b in range(B if "kv" not in _ABL else 0):
                cf = c_full[pl.ds(b * S, S), :]
                for n0 in range(0, D, KVT):
                    cols = pl.ds(n0, KVT)
                    k_ref[pl.ds(b * S, S), cols] = _mm(
                        cf, w_kv[0, :, cols]).astype(BF16)
                    v_ref[pl.ds(b * S, S), cols] = _mm(
                        cf, w_kv[1, :, cols]).astype(BF16)

        GH = 4
        NG = H // GH
        GD = GH * Dh

        def wo_cp(g):
            return pltpu.make_async_copy(
                wo_hbm.at[pl.ds(g * GD, GD), :],
                wo_buf.at[g % 2], wo_sems.at[g % 2])

        if "attnout" in _ABL:
            for b in range(B):
                out_ref[b] = jnp.zeros((S, D), jnp.float32)
        else:
            wo_cp(0).start()
        with jax.named_scope("phase#attnout"):
            for g in range(NG if "attnout" not in _ABL else 0):
                if g + 1 < NG:
                    wo_cp(g + 1).start()
                for hl in range(GH):
                    h = g * GH + hl
                    co = h * Dh
                    for b in range(B):
                        ro = b * S
                        q = q_ref[pl.ds(ro, S), pl.ds(co, Dh)]
                        k = k_ref[pl.ds(ro, S), pl.ds(co, Dh)]
                        qr = qr_ref[pl.ds(ro, S), h * Dr:(h + 1) * Dr]
                        kr = kr_ref[pl.ds(ro, S), :]
                        p = jnp.exp(_mmT(q, k) + _mmT(qr, kr))
                        r = 1.0 / jnp.sum(p, axis=1, keepdims=True)
                        o = _mm(p.astype(BF16),
                                v_ref[pl.ds(ro, S), pl.ds(co, Dh)])
                        ob_buf[pl.ds(ro, S), pl.ds(hl * Dh, Dh)] = (
                            o * r).astype(BF16)
                wo_cp(g).wait()
                for n0 in range(0, D, D // 2):
                    woh = wo_buf[g % 2][:, n0:n0 + D // 2].astype(BF16)
                    cols = pl.ds(n0, D // 2)
                    for b in range(B):
                        contrib = _mm(ob_buf[pl.ds(b * S, S), :], woh)
                        if g == 0:
                            out_ref[b, :, cols] = contrib
                        else:
                            out_ref[b, :, cols] = (
                                out_ref[b, :, cols] + contrib)

    vmem = pl.BlockSpec(memory_space=pltpu.VMEM)
    hbm = pl.BlockSpec(memory_space=pl.ANY)
    return pl.pallas_call(
        body,
        out_shape=jax.ShapeDtypeStruct((B, S, D), jnp.float32),
        in_specs=[vmem, vmem, vmem, vmem, hbm, vmem, vmem, hbm],
        out_specs=vmem,
        scratch_shapes=[
            pltpu.VMEM((BS, D), BF16),
            pltpu.VMEM((BS, 2 * DC), BF16),
            pltpu.VMEM((2, 2 * DC, D), BF16),
            pltpu.VMEM((BS, D), BF16),
            pltpu.VMEM((BS, H * Dr), BF16),
            pltpu.VMEM((BS, Dr), BF16),
            pltpu.VMEM((BS, D), BF16),
            pltpu.VMEM((BS, D), BF16),
            pltpu.VMEM((2, D, NT), jnp.float32),
            pltpu.VMEM((2, 4 * Dh, D), jnp.float32),
            pltpu.VMEM((BS, 4 * Dh), BF16),
            pltpu.SemaphoreType.DMA((2,)),
            pltpu.SemaphoreType.DMA((2,)),
            pltpu.SemaphoreType.DMA((2,)),
            pltpu.SemaphoreType.DMA((2,)),
        ],
        compiler_params=pltpu.CompilerParams(
            collective_id=0,
            vmem_limit_bytes=61 * 1024 * 1024,
        ),
    )(x, Wdkv, Wuk, Wuv, Wq, Wqr, Wkr, Wo)


# device time: 65350 ns/iter; 1.3093x vs baseline; 1.0269x over previous
import os

import jax
import jax.numpy as jnp
from jax import lax
from jax.experimental import pallas as pl
from jax.experimental.pallas import tpu as pltpu

B, S, H, Dh, Dr = 2, 512, 16, 128, 32
D = 2048
DC = 128
BS = B * S
SCALE = (Dh + Dr) ** -0.5
BF16 = jnp.bfloat16
_ABL = set(os.environ.get("KABL", "").split(","))
NT = 256
NSTRIP = D // NT
KVT = 512


def _mm(a, b):
    return lax.dot_general(a, b, (((1,), (0,)), ((), ())),
                           preferred_element_type=jnp.float32)


def _mmT(a, b):
    return lax.dot_general(a, b, (((1,), (1,)), ((), ())),
                           preferred_element_type=jnp.float32)


def kernel(x, Wdkv, Wuk, Wuv, Wq, Wqr, Wkr, Wo):
    def body(x_ref, wdkv_ref, wuk_ref, wuv_ref, wq_hbm, wqr_ref, wkr_ref,
             wo_hbm, out_ref,
             xb_ref, c_full, w_kv,
             q_ref, qr_ref, kr_ref, k_ref, v_ref, strip_buf, wo_buf,
             ob_buf, send_sems, recv_sems, strip_sems, wo_sems):
        my_x = lax.axis_index("x")
        my_y = lax.axis_index("y")
        my_z = lax.axis_index("z")
        partner = (1 - my_x, my_y, my_z)

        def strip_cp(hbm_ref, n):
            return pltpu.make_async_copy(
                hbm_ref.at[:, pl.ds(n * NT, NT)],
                strip_buf.at[n % 2], strip_sems.at[n % 2])

        if "qstream" not in _ABL:
            strip_cp(wq_hbm, 0).start()

        if "comm" not in _ABL:
            barrier = pltpu.get_barrier_semaphore()
            pl.semaphore_signal(barrier, inc=1, device_id=partner,
                                device_id_type=pl.DeviceIdType.MESH)
            pl.semaphore_wait(barrier, 1)

        with jax.named_scope("phase#send"):
            w_kv[0, :DC, :] = wuk_ref[:].astype(BF16)
            w_kv[1, :DC, :] = wuv_ref[:].astype(BF16)
            w_rdmas = []
            for g in range(4):
                gcols = pl.ds(g * 512, 512)
                w_rdmas.append(pltpu.make_async_remote_copy(
                    src_ref=w_kv.at[:, pl.ds(0, DC), gcols],
                    dst_ref=w_kv.at[:, pl.ds(DC, DC), gcols],
                    send_sem=send_sems.at[g], recv_sem=recv_sems.at[g],
                    device_id=partner,
                    device_id_type=pl.DeviceIdType.MESH))
            if "comm" not in _ABL:
                for r in w_rdmas:
                    r.start()

            wdkv = wdkv_ref[:].astype(BF16)
            for b in range(B):
                xb_ref[pl.ds(b * S, S), :] = x_ref[b].astype(BF16)
                c_full[pl.ds(b * S, S), :DC] = _mm(
                    xb_ref[pl.ds(b * S, S), :], wdkv).astype(BF16)
            c_rdma = pltpu.make_async_remote_copy(
                src_ref=c_full.at[:, pl.ds(0, DC)],
                dst_ref=c_full.at[:, pl.ds(DC, DC)],
                send_sem=send_sems.at[4], recv_sem=recv_sems.at[4],
                device_id=partner, device_id_type=pl.DeviceIdType.MESH)
            if "comm" not in _ABL:
                c_rdma.start()

        with jax.named_scope("phase#qstream"):
            for n in range(NSTRIP if "qstream" not in _ABL else 0):
                if n + 1 < NSTRIP:
                    strip_cp(wq_hbm, n + 1).start()
                strip_cp(wq_hbm, n).wait()
                wqs = strip_buf[n % 2].astype(BF16)
                for b in range(B):
                    q_ref[pl.ds(b * S, S), pl.ds(n * NT, NT)] = (
                        _mm(xb_ref[pl.ds(b * S, S), :], wqs) * SCALE
                    ).astype(BF16)

        with jax.named_scope("phase#qr_kr"):
            wqr = wqr_ref[:].astype(BF16)
            wkr = wkr_ref[:].astype(BF16)
            for b in range(B):
                xb = xb_ref[pl.ds(b * S, S), :]
                qr_ref[pl.ds(b * S, S), :] = (
                    _mm(xb, wqr) * SCALE).astype(BF16)
                kr_ref[pl.ds(b * S, S), :] = _mm(xb, wkr).astype(BF16)

        with jax.named_scope("phase#rdmawait"):
            if "comm" not in _ABL:
                c_rdma.wait()

        GH = 4
        NG = H // GH
        GD = GH * Dh

        def wo_cp(g):
            return pltpu.make_async_copy(
                wo_hbm.at[pl.ds(g * GD, GD), :],
                wo_buf.at[g % 2], wo_sems.at[g % 2])

        if "attnout" in _ABL:
            for b in range(B):
                out_ref[b] = jnp.zeros((S, D), jnp.float32)
        else:
            wo_cp(0).start()
        with jax.named_scope("phase#attnout"):
            for g in range(NG if "attnout" not in _ABL else 0):
                if g + 1 < NG:
                    wo_cp(g + 1).start()
                if "comm" not in _ABL:
                    w_rdmas[g].wait()
                gcols = pl.ds(g * GD, GD)
                for b in range(B if "kv" not in _ABL else 0):
                    cf = c_full[pl.ds(b * S, S), :]
                    k_ref[pl.ds(b * S, S), gcols] = _mm(
                        cf, w_kv[0, :, gcols]).astype(BF16)
                    v_ref[pl.ds(b * S, S), gcols] = _mm(
                        cf, w_kv[1, :, gcols]).astype(BF16)
                for hl in range(GH):
                    h = g * GH + hl
                    co = h * Dh
                    for b in range(B):
                        ro = b * S
                        q = q_ref[pl.ds(ro, S), pl.ds(co, Dh)]
                        k = k_ref[pl.ds(ro, S), pl.ds(co, Dh)]
                        qr = qr_ref[pl.ds(ro, S), h * Dr:(h + 1) * Dr]
                        kr = kr_ref[pl.ds(ro, S), :]
                        p = jnp.exp(_mmT(q, k) + _mmT(qr, kr))
                        r = 1.0 / jnp.sum(p, axis=1, keepdims=True)
                        o = _mm(p.astype(BF16),
                                v_ref[pl.ds(ro, S), pl.ds(co, Dh)])
                        ob_buf[pl.ds(ro, S), pl.ds(hl * Dh, Dh)] = (
                            o * r).astype(BF16)
                wo_cp(g).wait()
                for n0 in range(0, D, D // 2):
                    woh = wo_buf[g % 2][:, n0:n0 + D // 2].astype(BF16)
                    cols = pl.ds(n0, D // 2)
                    for b in range(B):
                        contrib = _mm(ob_buf[pl.ds(b * S, S), :], woh)
                        if g == 0:
                            out_ref[b, :, cols] = contrib
                        else:
                            out_ref[b, :, cols] = (
                                out_ref[b, :, cols] + contrib)

    vmem = pl.BlockSpec(memory_space=pltpu.VMEM)
    hbm = pl.BlockSpec(memory_space=pl.ANY)
    return pl.pallas_call(
        body,
        out_shape=jax.ShapeDtypeStruct((B, S, D), jnp.float32),
        in_specs=[vmem, vmem, vmem, vmem, hbm, vmem, vmem, hbm],
        out_specs=vmem,
        scratch_shapes=[
            pltpu.VMEM((BS, D), BF16),
            pltpu.VMEM((BS, 2 * DC), BF16),
            pltpu.VMEM((2, 2 * DC, D), BF16),
            pltpu.VMEM((BS, D), BF16),
            pltpu.VMEM((BS, H * Dr), BF16),
            pltpu.VMEM((BS, Dr), BF16),
            pltpu.VMEM((BS, D), BF16),
            pltpu.VMEM((BS, D), BF16),
            pltpu.VMEM((2, D, NT), jnp.float32),
            pltpu.VMEM((2, 4 * Dh, D), jnp.float32),
            pltpu.VMEM((BS, 4 * Dh), BF16),
            pltpu.SemaphoreType.DMA((5,)),
            pltpu.SemaphoreType.DMA((5,)),
            pltpu.SemaphoreType.DMA((2,)),
            pltpu.SemaphoreType.DMA((2,)),
        ],
        compiler_params=pltpu.CompilerParams(
            collective_id=0 if "comm" not in _ABL else None,
            vmem_limit_bytes=61 * 1024 * 1024,
        ),
    )(x, Wdkv, Wuk, Wuv, Wq, Wqr, Wkr, Wo)
